# Initial kernel scaffold; baseline (speedup 1.0000x reference)
import jax
import jax.numpy as jnp
from jax.experimental import pallas as pl


def kernel(x, edge_attr, W_rel1, b_rel1, W_root1, W_rel2, b_rel2, W_root2, edge_index):
    raise NotImplementedError("write your pallas kernel here")


# speedup vs baseline: 4.1040x; 4.1040x over previous
import functools

import jax
import jax.numpy as jnp
from jax import lax
from jax.experimental import pallas as pl
from jax.experimental.pallas import tpu as pltpu
from jax.experimental.pallas import tpu_sc as plsc

N = 10000
E = 160000
LANE = 128
E_PAD = 163840
IDX_ROWS = E_PAD // LANE
TILES = 16
ROWS_PER_TILE = IDX_ROWS // TILES
GRP = 8
N_GRP = ROWS_PER_TILE // GRP
STRIPE = 624
EXTRA = N - TILES * STRIPE
JUNK = 16
BM = 1000


def _make_segsum(n_chunks):
    half = n_chunks // 2
    mesh = plsc.VectorSubcoreMesh(core_axis_name="c", subcore_axis_name="s")

    @functools.partial(
        pl.kernel,
        mesh=mesh,
        out_type=[jax.ShapeDtypeStruct((N, LANE), jnp.float32)] * n_chunks,
        scratch_types=[
            pltpu.VMEM((GRP, LANE), jnp.int32),
            pltpu.VMEM((GRP, LANE), jnp.int32),
            pltpu.VMEM((LANE, LANE), jnp.float32),
            pltpu.VMEM_SHARED((N + JUNK, LANE), jnp.float32),
            pltpu.SemaphoreType.DMA,
        ],
    )
    def segsum(*refs):
        xs = refs[:n_chunks]
        src_r = refs[n_chunks]
        dst_r = refs[n_chunks + 1]
        zero_r = refs[n_chunks + 2]
        outs = refs[n_chunks + 3 : 2 * n_chunks + 3]
        sbuf, dbuf, rows, acc, sem = refs[2 * n_chunks + 3 :]

        c = lax.axis_index("c")
        s = lax.axis_index("s")

        def process(x_ref, out_ref):
            pltpu.sync_copy(zero_r, acc.at[pl.ds(s * STRIPE, STRIPE)])

            @pl.when(s == TILES - 1)
            def _():
                pltpu.sync_copy(
                    zero_r.at[pl.ds(0, EXTRA)],
                    acc.at[pl.ds(TILES * STRIPE, EXTRA)],
                )

            plsc.subcore_barrier()

            def grp(g, carry):
                rb = s * ROWS_PER_TILE + g * GRP
                pltpu.sync_copy(src_r.at[pl.ds(rb, GRP)], sbuf)
                pltpu.sync_copy(dst_r.at[pl.ds(rb, GRP)], dbuf)
                for j in range(GRP):
                    pltpu.async_copy(x_ref.at[sbuf.at[j]], rows, sem).wait()
                    pltpu.sync_copy(rows, acc.at[dbuf.at[j]], add=True)
                return carry

            lax.fori_loop(0, N_GRP, grp, 0)
            plsc.subcore_barrier()
            pltpu.sync_copy(
                acc.at[pl.ds(s * STRIPE, STRIPE)],
                out_ref.at[pl.ds(s * STRIPE, STRIPE)],
            )

            @pl.when(s == TILES - 1)
            def _():
                pltpu.sync_copy(
                    acc.at[pl.ds(TILES * STRIPE, EXTRA)],
                    out_ref.at[pl.ds(TILES * STRIPE, EXTRA)],
                )

        for cid in range(2):
            for j in range(half):
                ch = cid * half + j

                @pl.when(c == cid)
                def _(ch=ch):
                    process(xs[ch], outs[ch])

    return segsum


def _fused_matmul(lhs_list, rhs_list, bias, relu, out_chunks):
    n_in = len(lhs_list)
    m = lhs_list[0].shape[0]
    n_out = rhs_list[0].shape[1]
    grid = (m // BM,)

    def body(*refs):
        ls = refs[:n_in]
        rs = refs[n_in : 2 * n_in]
        b = refs[2 * n_in]
        outs = refs[2 * n_in + 1 :]
        acc = b[...]
        for i in range(n_in):
            acc = acc + jnp.dot(
                ls[i][...],
                rs[i][...],
                preferred_element_type=jnp.float32,
                precision=lax.Precision.HIGHEST,
            )
        if relu:
            acc = jnp.maximum(acc, 0.0)
        if out_chunks == 1:
            outs[0][...] = acc
        else:
            for i in range(out_chunks):
                outs[i][...] = acc[:, i * LANE : (i + 1) * LANE]

    in_specs = (
        [
            pl.BlockSpec((BM, l.shape[1]), lambda mi: (mi, 0))
            for l in lhs_list
        ]
        + [pl.BlockSpec(r.shape, lambda mi: (0, 0)) for r in rhs_list]
        + [pl.BlockSpec((1, n_out), lambda mi: (0, 0))]
    )
    if out_chunks == 1:
        out_shape = jax.ShapeDtypeStruct((m, n_out), jnp.float32)
        out_specs = pl.BlockSpec((BM, n_out), lambda mi: (mi, 0))
    else:
        out_shape = [
            jax.ShapeDtypeStruct((m, LANE), jnp.float32)
            for _ in range(out_chunks)
        ]
        out_specs = [
            pl.BlockSpec((BM, LANE), lambda mi: (mi, 0))
            for _ in range(out_chunks)
        ]
    return pl.pallas_call(
        body,
        grid=grid,
        in_specs=in_specs,
        out_specs=out_specs,
        out_shape=out_shape,
    )(*lhs_list, *rhs_list, bias.reshape(1, n_out))


def kernel(x, edge_attr, W_rel1, b_rel1, W_root1, W_rel2, b_rel2, W_root2,
           edge_index):
    del edge_attr
    src = edge_index[0].astype(jnp.int32)
    dst = edge_index[1].astype(jnp.int32)
    pad = E_PAD - E
    pad_i = jnp.arange(pad, dtype=jnp.int32)
    src_p = jnp.concatenate([src, pad_i % N]).reshape(IDX_ROWS, LANE)
    dst_p = jnp.concatenate([dst, N + (pad_i % JUNK)]).reshape(IDX_ROWS, LANE)
    zeros = jnp.zeros((STRIPE, LANE), jnp.float32)

    x0 = x[:, :LANE]
    x1 = x[:, LANE:]
    a0, a1 = _make_segsum(2)(x0, x1, src_p, dst_p, zeros)
    Wr1 = W_rel1.T
    h_chunks = _fused_matmul(
        [a0, a1, x],
        [Wr1[:LANE], Wr1[LANE:], W_root1.T],
        b_rel1,
        relu=True,
        out_chunks=4,
    )

    b_chunks = _make_segsum(4)(*h_chunks, src_p, dst_p, zeros)
    Wr2 = W_rel2.T
    Wt2 = W_root2.T
    out = _fused_matmul(
        list(b_chunks) + list(h_chunks),
        [Wr2[i * LANE : (i + 1) * LANE] for i in range(4)]
        + [Wt2[i * LANE : (i + 1) * LANE] for i in range(4)],
        b_rel2,
        relu=False,
        out_chunks=1,
    )
    return out


# speedup vs baseline: 4.8532x; 1.1826x over previous
import functools

import jax
import jax.numpy as jnp
from jax import lax
from jax.experimental import pallas as pl
from jax.experimental.pallas import tpu as pltpu
from jax.experimental.pallas import tpu_sc as plsc

N = 10000
E = 160000
LANE = 128
E_PAD = 163840
IDX_ROWS = E_PAD // LANE
TILES = 16
ROWS_PER_TILE = IDX_ROWS // TILES
NBUF = 2
HALF = ROWS_PER_TILE // 2
STRIPE = 624
EXTRA = N - TILES * STRIPE
JUNK = 16
BM = 1000


def _make_segsum(n_chunks):
    half = n_chunks // 2
    mesh = plsc.VectorSubcoreMesh(core_axis_name="c", subcore_axis_name="s")

    @functools.partial(
        pl.kernel,
        mesh=mesh,
        out_type=[jax.ShapeDtypeStruct((N, LANE), jnp.float32)] * n_chunks,
        scratch_types=[
            pltpu.VMEM((HALF, LANE), jnp.int32),
            pltpu.VMEM((HALF, LANE), jnp.int32),
            pltpu.VMEM((NBUF, LANE, LANE), jnp.float32),
            pltpu.VMEM_SHARED((N + JUNK, LANE), jnp.float32),
            pltpu.SemaphoreType.DMA((NBUF,)),
            pltpu.SemaphoreType.DMA((NBUF,)),
        ],
    )
    def segsum(*refs):
        xs = refs[:n_chunks]
        src_r = refs[n_chunks]
        dst_r = refs[n_chunks + 1]
        zero_r = refs[n_chunks + 2]
        outs = refs[n_chunks + 3 : 2 * n_chunks + 3]
        sbuf, dbuf, rows, acc, gsem, ssem = refs[2 * n_chunks + 3 :]

        c = lax.axis_index("c")
        s = lax.axis_index("s")

        n_grp = HALF // NBUF

        def process(x_ref, out_ref):
            pltpu.sync_copy(zero_r, acc.at[pl.ds(s * STRIPE, STRIPE)])

            @pl.when(s == TILES - 1)
            def _():
                pltpu.sync_copy(
                    zero_r.at[pl.ds(0, EXTRA)],
                    acc.at[pl.ds(TILES * STRIPE, EXTRA)],
                )

            plsc.subcore_barrier()

            def gather(j, b):
                pltpu.async_copy(
                    x_ref.at[sbuf.at[j]], rows.at[b], gsem.at[b]
                )

            def gather_wait(j, b):
                pltpu.make_async_copy(
                    x_ref.at[sbuf.at[j]], rows.at[b], gsem.at[b]
                ).wait()

            def scatter(j, b):
                pltpu.async_copy(
                    rows.at[b], acc.at[dbuf.at[j]], ssem.at[b], add=True
                )

            def scatter_wait(j, b):
                pltpu.make_async_copy(
                    rows.at[b], acc.at[dbuf.at[j]], ssem.at[b]
                ).wait()

            for h in range(2):
                rb = s * ROWS_PER_TILE + h * HALF
                pltpu.sync_copy(src_r.at[pl.ds(rb, HALF)], sbuf)
                pltpu.sync_copy(dst_r.at[pl.ds(rb, HALF)], dbuf)
                for b in range(NBUF):
                    gather(b, b)

                def grp(g, carry):
                    j0 = g * NBUF
                    for b in range(NBUF):
                        gather_wait(j0 + b, b)
                        scatter(j0 + b, b)

                    @pl.when(g < n_grp - 1)
                    def _():
                        for b in range(NBUF):
                            scatter_wait(j0 + b, b)
                            gather(j0 + NBUF + b, b)

                    return carry

                lax.fori_loop(0, n_grp, grp, 0)
                for b in range(NBUF):
                    scatter_wait((n_grp - 1) * NBUF + b, b)
            plsc.subcore_barrier()
            pltpu.sync_copy(
                acc.at[pl.ds(s * STRIPE, STRIPE)],
                out_ref.at[pl.ds(s * STRIPE, STRIPE)],
            )

            @pl.when(s == TILES - 1)
            def _():
                pltpu.sync_copy(
                    acc.at[pl.ds(TILES * STRIPE, EXTRA)],
                    out_ref.at[pl.ds(TILES * STRIPE, EXTRA)],
                )

        for cid in range(2):
            for j in range(half):
                ch = cid * half + j

                @pl.when(c == cid)
                def _(ch=ch):
                    process(xs[ch], outs[ch])

    return segsum


def _fused_matmul(lhs_list, rhs_list, bias, relu, out_chunks):
    n_in = len(lhs_list)
    m = lhs_list[0].shape[0]
    n_out = rhs_list[0].shape[1]
    grid = (m // BM,)

    def body(*refs):
        ls = refs[:n_in]
        rs = refs[n_in : 2 * n_in]
        b = refs[2 * n_in]
        outs = refs[2 * n_in + 1 :]
        acc = b[...]
        for i in range(n_in):
            acc = acc + jnp.dot(
                ls[i][...],
                rs[i][...],
                preferred_element_type=jnp.float32,
                precision=lax.Precision.HIGHEST,
            )
        if relu:
            acc = jnp.maximum(acc, 0.0)
        if out_chunks == 1:
            outs[0][...] = acc
        else:
            for i in range(out_chunks):
                outs[i][...] = acc[:, i * LANE : (i + 1) * LANE]

    in_specs = (
        [
            pl.BlockSpec((BM, l.shape[1]), lambda mi: (mi, 0))
            for l in lhs_list
        ]
        + [pl.BlockSpec(r.shape, lambda mi: (0, 0)) for r in rhs_list]
        + [pl.BlockSpec((1, n_out), lambda mi: (0, 0))]
    )
    if out_chunks == 1:
        out_shape = jax.ShapeDtypeStruct((m, n_out), jnp.float32)
        out_specs = pl.BlockSpec((BM, n_out), lambda mi: (mi, 0))
    else:
        out_shape = [
            jax.ShapeDtypeStruct((m, LANE), jnp.float32)
            for _ in range(out_chunks)
        ]
        out_specs = [
            pl.BlockSpec((BM, LANE), lambda mi: (mi, 0))
            for _ in range(out_chunks)
        ]
    return pl.pallas_call(
        body,
        grid=grid,
        in_specs=in_specs,
        out_specs=out_specs,
        out_shape=out_shape,
    )(*lhs_list, *rhs_list, bias.reshape(1, n_out))


def kernel(x, edge_attr, W_rel1, b_rel1, W_root1, W_rel2, b_rel2, W_root2,
           edge_index):
    del edge_attr
    src = edge_index[0].astype(jnp.int32)
    dst = edge_index[1].astype(jnp.int32)
    pad = E_PAD - E
    pad_i = jnp.arange(pad, dtype=jnp.int32)
    src_p = jnp.concatenate([src, pad_i % N]).reshape(IDX_ROWS, LANE)
    dst_p = jnp.concatenate([dst, N + (pad_i % JUNK)]).reshape(IDX_ROWS, LANE)
    zeros = jnp.zeros((STRIPE, LANE), jnp.float32)

    x0 = x[:, :LANE]
    x1 = x[:, LANE:]
    a0, a1 = _make_segsum(2)(x0, x1, src_p, dst_p, zeros)
    Wr1 = W_rel1.T
    h_chunks = _fused_matmul(
        [a0, a1, x],
        [Wr1[:LANE], Wr1[LANE:], W_root1.T],
        b_rel1,
        relu=True,
        out_chunks=4,
    )

    b_chunks = _make_segsum(4)(*h_chunks, src_p, dst_p, zeros)
    Wr2 = W_rel2.T
    Wt2 = W_root2.T
    out = _fused_matmul(
        list(b_chunks) + list(h_chunks),
        [Wr2[i * LANE : (i + 1) * LANE] for i in range(4)]
        + [Wt2[i * LANE : (i + 1) * LANE] for i in range(4)],
        b_rel2,
        relu=False,
        out_chunks=1,
    )
    return out


# speedup vs baseline: 5.8055x; 1.1962x over previous
import functools

import jax
import jax.numpy as jnp
from jax import lax
from jax.experimental import pallas as pl
from jax.experimental.pallas import tpu as pltpu
from jax.experimental.pallas import tpu_sc as plsc

N = 10000
E = 160000
LANE = 128
E_PAD = 163840
IDX_ROWS = E_PAD // LANE
TILES = 16
ROWS_PER_TILE = IDX_ROWS // TILES
HALF = ROWS_PER_TILE // 2
NBUF = 2
STRIPE = 624
EXTRA = N - TILES * STRIPE
JUNK = 8
BM = 1000
VEC = 16


def _make_segsum(n_chunks):
    half_ch = n_chunks // 2
    mesh = plsc.VectorSubcoreMesh(core_axis_name="c", subcore_axis_name="s")

    @functools.partial(
        pl.kernel,
        mesh=mesh,
        out_type=[jax.ShapeDtypeStruct((N, LANE), jnp.float32)] * n_chunks,
        scratch_types=[
            pltpu.VMEM((HALF, LANE), jnp.int32),
            pltpu.VMEM((HALF, LANE), jnp.int32),
            pltpu.VMEM((HALF, LANE), jnp.int32),
            pltpu.VMEM((NBUF, LANE, LANE), jnp.float32),
            pltpu.VMEM_SHARED((N + JUNK, LANE), jnp.float32),
            pltpu.SemaphoreType.DMA((NBUF,)),
            pltpu.SemaphoreType.DMA((NBUF,)),
        ],
    )
    def segsum(xf, src_r, dst_r, zero_r, *rest):
        outs = rest[:n_chunks]
        sbuf, sbuf2, dbuf, rows, acc, gsem, ssem = rest[n_chunks:]

        c = lax.axis_index("c")
        s = lax.axis_index("s")

        n_grp = HALF // NBUF

        def process(ch, out_ref):
            pltpu.sync_copy(zero_r, acc.at[pl.ds(s * STRIPE, STRIPE)])

            @pl.when(s == TILES - 1)
            def _():
                pltpu.sync_copy(
                    zero_r.at[pl.ds(0, EXTRA)],
                    acc.at[pl.ds(TILES * STRIPE, EXTRA)],
                )

            plsc.subcore_barrier()

            def gather(j, b):
                pltpu.async_copy(
                    xf.at[sbuf2.at[j]], rows.at[b], gsem.at[b]
                )

            def gather_wait(j, b):
                pltpu.make_async_copy(
                    xf.at[sbuf2.at[j]], rows.at[b], gsem.at[b]
                ).wait()

            def scatter(j, b):
                pltpu.async_copy(
                    rows.at[b], acc.at[dbuf.at[j]], ssem.at[b], add=True
                )

            def scatter_wait(j, b):
                pltpu.make_async_copy(
                    rows.at[b], acc.at[dbuf.at[j]], ssem.at[b]
                ).wait()

            for h in range(2):
                rb = s * ROWS_PER_TILE + h * HALF
                pltpu.sync_copy(src_r.at[pl.ds(rb, HALF)], sbuf)
                pltpu.sync_copy(dst_r.at[pl.ds(rb, HALF)], dbuf)

                def idxt(r, carry):
                    for v in range(LANE // VEC):
                        sl = pl.ds(v * VEC, VEC)
                        sbuf2[r, sl] = sbuf[r, sl] * n_chunks + ch
                    return carry

                lax.fori_loop(0, HALF, idxt, 0)

                for b in range(NBUF):
                    gather(b, b)

                def grp(g, carry):
                    j0 = g * NBUF
                    for b in range(NBUF):
                        gather_wait(j0 + b, b)
                        scatter(j0 + b, b)

                    @pl.when(g < n_grp - 1)
                    def _():
                        for b in range(NBUF):
                            scatter_wait(j0 + b, b)
                            gather(j0 + NBUF + b, b)

                    return carry

                lax.fori_loop(0, n_grp, grp, 0)
                for b in range(NBUF):
                    scatter_wait((n_grp - 1) * NBUF + b, b)
            plsc.subcore_barrier()
            pltpu.sync_copy(
                acc.at[pl.ds(s * STRIPE, STRIPE)],
                out_ref.at[pl.ds(s * STRIPE, STRIPE)],
            )

            @pl.when(s == TILES - 1)
            def _():
                pltpu.sync_copy(
                    acc.at[pl.ds(TILES * STRIPE, EXTRA)],
                    out_ref.at[pl.ds(TILES * STRIPE, EXTRA)],
                )

        for cid in range(2):
            for j in range(half_ch):
                ch = cid * half_ch + j

                @pl.when(c == cid)
                def _(ch=ch):
                    process(ch, outs[ch])

    return segsum


def _fused_matmul(lhs_list, rhs_list, bias, residual, relu):
    n_in = len(lhs_list)
    m = lhs_list[0].shape[0]
    n_out = rhs_list[0].shape[1]
    grid = (m // BM,)
    has_res = residual is not None

    def body(*refs):
        ls = refs[:n_in]
        rs = refs[n_in : 2 * n_in]
        b = refs[2 * n_in]
        res = refs[2 * n_in + 1] if has_res else None
        out = refs[-1]
        acc = b[...]
        if has_res:
            acc = acc + res[...]
        for i in range(n_in):
            acc = acc + jnp.dot(
                ls[i][...], rs[i][...], preferred_element_type=jnp.float32
            )
        if relu:
            acc = jnp.maximum(acc, 0.0)
        out[...] = acc

    in_specs = (
        [pl.BlockSpec((BM, l.shape[1]), lambda mi: (mi, 0)) for l in lhs_list]
        + [pl.BlockSpec(r.shape, lambda mi: (0, 0)) for r in rhs_list]
        + [pl.BlockSpec((1, n_out), lambda mi: (0, 0))]
    )
    args = list(lhs_list) + list(rhs_list) + [bias.reshape(1, n_out)]
    if has_res:
        in_specs.append(pl.BlockSpec((BM, n_out), lambda mi: (mi, 0)))
        args.append(residual)
    return pl.pallas_call(
        body,
        grid=grid,
        in_specs=in_specs,
        out_specs=pl.BlockSpec((BM, n_out), lambda mi: (mi, 0)),
        out_shape=jax.ShapeDtypeStruct((m, n_out), jnp.float32),
    )(*args)


def kernel(x, edge_attr, W_rel1, b_rel1, W_root1, W_rel2, b_rel2, W_root2,
           edge_index):
    del edge_attr
    src = edge_index[0].astype(jnp.int32)
    dst = edge_index[1].astype(jnp.int32)
    pad = E_PAD - E
    pad_i = jnp.arange(pad, dtype=jnp.int32)
    src_p = jnp.concatenate([src, pad_i % N]).reshape(IDX_ROWS, LANE)
    dst_p = jnp.concatenate([dst, N + (pad_i % JUNK)]).reshape(IDX_ROWS, LANE)
    zeros = jnp.zeros((STRIPE, LANE), jnp.float32)

    r1 = _fused_matmul([x], [W_root1.T], b_rel1, None, relu=False)
    a0, a1 = _make_segsum(2)(
        x.reshape(2 * N, LANE), src_p, dst_p, zeros
    )
    Wr1 = W_rel1.T
    h = _fused_matmul([a0, a1], [Wr1[:LANE], Wr1[LANE:]], b_rel1 * 0.0, r1,
                      relu=True)

    r2 = _fused_matmul([h], [W_root2.T], b_rel2, None, relu=False)
    b_chunks = _make_segsum(4)(
        h.reshape(4 * N, LANE), src_p, dst_p, zeros
    )
    Wr2 = W_rel2.T
    out = _fused_matmul(
        list(b_chunks),
        [Wr2[i * LANE : (i + 1) * LANE] for i in range(4)],
        b_rel2 * 0.0,
        r2,
        relu=False,
    )
    return out


# speedup vs baseline: 6.0149x; 1.0361x over previous
import functools

import jax
import jax.numpy as jnp
from jax import lax
from jax.experimental import pallas as pl
from jax.experimental.pallas import tpu as pltpu
from jax.experimental.pallas import tpu_sc as plsc

N = 10000
E = 160000
LANE = 128
E_PAD = 163840
IDX_ROWS = E_PAD // LANE
TILES = 16
ROWS_PER_TILE = IDX_ROWS // TILES
HALF = ROWS_PER_TILE // 2
NBUF = 2
STRIPE = 624
EXTRA = N - TILES * STRIPE
JUNK = 8
BM = 1000
VEC = 16


def _make_segsum(n_chunks):
    half_ch = n_chunks // 2
    mesh = plsc.VectorSubcoreMesh(core_axis_name="c", subcore_axis_name="s")

    @functools.partial(
        pl.kernel,
        mesh=mesh,
        out_type=[jax.ShapeDtypeStruct((N, LANE), jnp.float32)] * n_chunks,
        scratch_types=[
            pltpu.VMEM((HALF, LANE), jnp.int32),
            pltpu.VMEM((HALF, LANE), jnp.int32),
            pltpu.VMEM((HALF, LANE), jnp.int32),
            pltpu.VMEM((NBUF, LANE, LANE), jnp.float32),
            pltpu.VMEM_SHARED((N + JUNK, LANE), jnp.float32),
            pltpu.SemaphoreType.DMA((NBUF,)),
            pltpu.SemaphoreType.DMA((NBUF,)),
        ],
    )
    def segsum(xf, src_r, dst_r, zero_r, *rest):
        outs = rest[:n_chunks]
        sbuf, sbuf2, dbuf, rows, acc, gsem, ssem = rest[n_chunks:]

        c = lax.axis_index("c")
        s = lax.axis_index("s")

        n_grp = HALF // NBUF

        def process(ch, out_ref):
            pltpu.sync_copy(zero_r, acc.at[pl.ds(s * STRIPE, STRIPE)])

            @pl.when(s == TILES - 1)
            def _():
                pltpu.sync_copy(
                    zero_r.at[pl.ds(0, EXTRA)],
                    acc.at[pl.ds(TILES * STRIPE, EXTRA)],
                )

            plsc.subcore_barrier()

            def gather(j, b):
                pltpu.async_copy(
                    xf.at[sbuf2.at[j]], rows.at[b], gsem.at[b]
                )

            def gather_wait(j, b):
                pltpu.make_async_copy(
                    xf.at[sbuf2.at[j]], rows.at[b], gsem.at[b]
                ).wait()

            def scatter(j, b):
                pltpu.async_copy(
                    rows.at[b], acc.at[dbuf.at[j]], ssem.at[b], add=True
                )

            def scatter_wait(j, b):
                pltpu.make_async_copy(
                    rows.at[b], acc.at[dbuf.at[j]], ssem.at[b]
                ).wait()

            for h in range(2):
                rb = s * ROWS_PER_TILE + h * HALF
                pltpu.sync_copy(src_r.at[pl.ds(rb, HALF)], sbuf)
                pltpu.sync_copy(dst_r.at[pl.ds(rb, HALF)], dbuf)

                def idxt(r, carry):
                    for v in range(LANE // VEC):
                        sl = pl.ds(v * VEC, VEC)
                        sbuf2[r, sl] = sbuf[r, sl] * n_chunks + ch
                    return carry

                lax.fori_loop(0, HALF, idxt, 0)

                for b in range(NBUF):
                    gather(b, b)

                def grp(g, carry):
                    j0 = g * NBUF
                    for b in range(NBUF):
                        gather_wait(j0 + b, b)
                        scatter(j0 + b, b)

                    @pl.when(g < n_grp - 1)
                    def _():
                        for b in range(NBUF):
                            scatter_wait(j0 + b, b)
                            gather(j0 + NBUF + b, b)

                    return carry

                lax.fori_loop(0, n_grp, grp, 0)
                for b in range(NBUF):
                    scatter_wait((n_grp - 1) * NBUF + b, b)
            plsc.subcore_barrier()
            pltpu.sync_copy(
                acc.at[pl.ds(s * STRIPE, STRIPE)],
                out_ref.at[pl.ds(s * STRIPE, STRIPE)],
            )

            @pl.when(s == TILES - 1)
            def _():
                pltpu.sync_copy(
                    acc.at[pl.ds(TILES * STRIPE, EXTRA)],
                    out_ref.at[pl.ds(TILES * STRIPE, EXTRA)],
                )

        for cid in range(2):
            for j in range(half_ch):
                ch = cid * half_ch + j

                @pl.when(c == cid)
                def _(ch=ch):
                    process(ch, outs[ch])

    return segsum


def _fused_matmul(lhs_list, rhs_list, bias, residual, relu,
                  lhs_fold=None, out_fold=1):
    n_in = len(lhs_list)
    lhs_fold = lhs_fold or [1] * n_in
    m = lhs_list[0].shape[0] // lhs_fold[0]
    n_out = rhs_list[0].shape[1]
    grid = (m // BM,)
    has_res = residual is not None

    def body(*refs):
        ls = refs[:n_in]
        rs = refs[n_in : 2 * n_in]
        b = refs[2 * n_in]
        res = refs[2 * n_in + 1] if has_res else None
        out = refs[-1]
        acc = b[...]
        if has_res:
            acc = acc + res[...]
        for i in range(n_in):
            lv = ls[i][...]
            if lhs_fold[i] > 1:
                lv = lv.reshape(BM, LANE * lhs_fold[i])
            acc = acc + jnp.dot(
                lv, rs[i][...], preferred_element_type=jnp.float32
            )
        if relu:
            acc = jnp.maximum(acc, 0.0)
        if out_fold > 1:
            out[...] = acc.reshape(out_fold * BM, LANE)
        else:
            out[...] = acc

    in_specs = (
        [
            pl.BlockSpec((f * BM, l.shape[1]), lambda mi: (mi, 0))
            for l, f in zip(lhs_list, lhs_fold)
        ]
        + [pl.BlockSpec(r.shape, lambda mi: (0, 0)) for r in rhs_list]
        + [pl.BlockSpec((1, n_out), lambda mi: (0, 0))]
    )
    args = list(lhs_list) + list(rhs_list) + [bias.reshape(1, n_out)]
    if has_res:
        in_specs.append(pl.BlockSpec((BM, n_out), lambda mi: (mi, 0)))
        args.append(residual)
    if out_fold > 1:
        out_specs = pl.BlockSpec((out_fold * BM, LANE), lambda mi: (mi, 0))
        out_shape = jax.ShapeDtypeStruct((out_fold * m, LANE), jnp.float32)
    else:
        out_specs = pl.BlockSpec((BM, n_out), lambda mi: (mi, 0))
        out_shape = jax.ShapeDtypeStruct((m, n_out), jnp.float32)
    return pl.pallas_call(
        body,
        grid=grid,
        in_specs=in_specs,
        out_specs=out_specs,
        out_shape=out_shape,
    )(*args)


def kernel(x, edge_attr, W_rel1, b_rel1, W_root1, W_rel2, b_rel2, W_root2,
           edge_index):
    del edge_attr
    src = edge_index[0].astype(jnp.int32)
    dst = edge_index[1].astype(jnp.int32)
    pad = E_PAD - E
    pad_i = jnp.arange(pad, dtype=jnp.int32)
    src_p = jnp.concatenate([src, pad_i % N]).reshape(IDX_ROWS, LANE)
    dst_p = jnp.concatenate([dst, N + (pad_i % JUNK)]).reshape(IDX_ROWS, LANE)
    zeros = jnp.zeros((STRIPE, LANE), jnp.float32)

    r1 = _fused_matmul([x], [W_root1.T], b_rel1, None, relu=False)
    a0, a1 = _make_segsum(2)(x.reshape(2 * N, LANE), src_p, dst_p, zeros)
    Wr1 = W_rel1.T
    h4 = _fused_matmul([a0, a1], [Wr1[:LANE], Wr1[LANE:]], b_rel1 * 0.0, r1,
                       relu=True, out_fold=4)

    r2 = _fused_matmul([h4], [W_root2.T], b_rel2, None, relu=False,
                       lhs_fold=[4])
    b_chunks = _make_segsum(4)(h4, src_p, dst_p, zeros)
    Wr2 = W_rel2.T
    out = _fused_matmul(
        list(b_chunks),
        [Wr2[i * LANE : (i + 1) * LANE] for i in range(4)],
        b_rel2 * 0.0,
        r2,
        relu=False,
    )
    return out


# speedup vs baseline: 6.5215x; 1.0842x over previous
import functools

import jax
import jax.numpy as jnp
from jax import lax
from jax.experimental import pallas as pl
from jax.experimental.pallas import tpu as pltpu
from jax.experimental.pallas import tpu_sc as plsc

N = 10000
E = 160000
LANE = 128
E_PAD = 163840
IDX_ROWS = E_PAD // LANE
TILES = 16
ROWS_PER_TILE = IDX_ROWS // TILES
HALF = ROWS_PER_TILE // 2
NBUF = 2
STRIPE = 624
EXTRA = N - TILES * STRIPE
JUNK = 8
BM = 1000
VEC = 16


def _make_segsum(n_chunks):
    half_ch = n_chunks // 2
    mesh = plsc.VectorSubcoreMesh(core_axis_name="c", subcore_axis_name="s")

    @functools.partial(
        pl.kernel,
        mesh=mesh,
        out_type=[jax.ShapeDtypeStruct((N, LANE), jnp.float32)] * n_chunks,
        scratch_types=[
            pltpu.VMEM((HALF, LANE), jnp.int32),
            pltpu.VMEM((HALF, LANE), jnp.int32),
            pltpu.VMEM((HALF, LANE), jnp.int32),
            pltpu.VMEM((NBUF, LANE, LANE), jnp.float32),
            pltpu.VMEM_SHARED((N + JUNK, LANE), jnp.float32),
            pltpu.SemaphoreType.DMA((NBUF,)),
            pltpu.SemaphoreType.DMA((NBUF,)),
        ],
    )
    def segsum(xf, src_r, dst_r, zero_r, *rest):
        outs = rest[:n_chunks]
        sbuf, sbuf2, dbuf, rows, acc, gsem, ssem = rest[n_chunks:]

        c = lax.axis_index("c")
        s = lax.axis_index("s")

        n_grp = HALF // NBUF

        def process(ch, out_ref):
            pltpu.sync_copy(zero_r, acc.at[pl.ds(s * STRIPE, STRIPE)])

            @pl.when(s == TILES - 1)
            def _():
                pltpu.sync_copy(
                    zero_r.at[pl.ds(0, EXTRA)],
                    acc.at[pl.ds(TILES * STRIPE, EXTRA)],
                )

            plsc.subcore_barrier()

            def gather(j, b):
                pltpu.async_copy(
                    xf.at[sbuf2.at[j]], rows.at[b], gsem.at[b]
                )

            def gather_wait(j, b):
                pltpu.make_async_copy(
                    xf.at[sbuf2.at[j]], rows.at[b], gsem.at[b]
                ).wait()

            def scatter(j, b):
                pltpu.async_copy(
                    rows.at[b], acc.at[dbuf.at[j]], ssem.at[b], add=True
                )

            def scatter_wait(j, b):
                pltpu.make_async_copy(
                    rows.at[b], acc.at[dbuf.at[j]], ssem.at[b]
                ).wait()

            for h in range(2):
                rb = s * ROWS_PER_TILE + h * HALF
                pltpu.sync_copy(src_r.at[pl.ds(rb, HALF)], sbuf)
                pltpu.sync_copy(dst_r.at[pl.ds(rb, HALF)], dbuf)

                def idxt(r, carry):
                    for v in range(LANE // VEC):
                        sl = pl.ds(v * VEC, VEC)
                        sbuf2[r, sl] = sbuf[r, sl] * n_chunks + ch
                    return carry

                lax.fori_loop(0, HALF, idxt, 0)

                gather(0, 0)

                def grp(g, carry):
                    j0 = 2 * g

                    gather_wait(j0, 0)
                    scatter(j0, 0)

                    @pl.when(g > 0)
                    def _():
                        scatter_wait(j0 - 1, 1)

                    gather(j0 + 1, 1)

                    gather_wait(j0 + 1, 1)
                    scatter(j0 + 1, 1)
                    scatter_wait(j0, 0)

                    @pl.when(g < n_grp - 1)
                    def _():
                        gather(j0 + 2, 0)

                    return carry

                lax.fori_loop(0, n_grp, grp, 0)
                scatter_wait(HALF - 1, 1)
            plsc.subcore_barrier()
            pltpu.sync_copy(
                acc.at[pl.ds(s * STRIPE, STRIPE)],
                out_ref.at[pl.ds(s * STRIPE, STRIPE)],
            )

            @pl.when(s == TILES - 1)
            def _():
                pltpu.sync_copy(
                    acc.at[pl.ds(TILES * STRIPE, EXTRA)],
                    out_ref.at[pl.ds(TILES * STRIPE, EXTRA)],
                )

        for cid in range(2):
            for j in range(half_ch):
                ch = cid * half_ch + j

                @pl.when(c == cid)
                def _(ch=ch):
                    process(ch, outs[ch])

    return segsum


def _fused_matmul(lhs_list, rhs_list, bias, residual, relu,
                  lhs_fold=None, out_fold=1):
    n_in = len(lhs_list)
    lhs_fold = lhs_fold or [1] * n_in
    m = lhs_list[0].shape[0] // lhs_fold[0]
    n_out = rhs_list[0].shape[1]
    grid = (m // BM,)
    has_res = residual is not None

    def body(*refs):
        ls = refs[:n_in]
        rs = refs[n_in : 2 * n_in]
        b = refs[2 * n_in]
        res = refs[2 * n_in + 1] if has_res else None
        out = refs[-1]
        acc = b[...]
        if has_res:
            acc = acc + res[...]
        for i in range(n_in):
            lv = ls[i][...]
            if lhs_fold[i] > 1:
                lv = lv.reshape(BM, LANE * lhs_fold[i])
            acc = acc + jnp.dot(
                lv, rs[i][...], preferred_element_type=jnp.float32
            )
        if relu:
            acc = jnp.maximum(acc, 0.0)
        if out_fold > 1:
            out[...] = acc.reshape(out_fold * BM, LANE)
        else:
            out[...] = acc

    in_specs = (
        [
            pl.BlockSpec((f * BM, l.shape[1]), lambda mi: (mi, 0))
            for l, f in zip(lhs_list, lhs_fold)
        ]
        + [pl.BlockSpec(r.shape, lambda mi: (0, 0)) for r in rhs_list]
        + [pl.BlockSpec((1, n_out), lambda mi: (0, 0))]
    )
    args = list(lhs_list) + list(rhs_list) + [bias.reshape(1, n_out)]
    if has_res:
        in_specs.append(pl.BlockSpec((BM, n_out), lambda mi: (mi, 0)))
        args.append(residual)
    if out_fold > 1:
        out_specs = pl.BlockSpec((out_fold * BM, LANE), lambda mi: (mi, 0))
        out_shape = jax.ShapeDtypeStruct((out_fold * m, LANE), jnp.float32)
    else:
        out_specs = pl.BlockSpec((BM, n_out), lambda mi: (mi, 0))
        out_shape = jax.ShapeDtypeStruct((m, n_out), jnp.float32)
    return pl.pallas_call(
        body,
        grid=grid,
        in_specs=in_specs,
        out_specs=out_specs,
        out_shape=out_shape,
    )(*args)


def kernel(x, edge_attr, W_rel1, b_rel1, W_root1, W_rel2, b_rel2, W_root2,
           edge_index):
    del edge_attr
    src = edge_index[0].astype(jnp.int32)
    dst = edge_index[1].astype(jnp.int32)
    pad = E_PAD - E
    pad_i = jnp.arange(pad, dtype=jnp.int32)
    src_p = jnp.concatenate([src, pad_i % N]).reshape(IDX_ROWS, LANE)
    dst_p = jnp.concatenate([dst, N + (pad_i % JUNK)]).reshape(IDX_ROWS, LANE)
    zeros = jnp.zeros((STRIPE, LANE), jnp.float32)

    r1 = _fused_matmul([x], [W_root1.T], b_rel1, None, relu=False)
    a0, a1 = _make_segsum(2)(x.reshape(2 * N, LANE), src_p, dst_p, zeros)
    Wr1 = W_rel1.T
    h4 = _fused_matmul([a0, a1], [Wr1[:LANE], Wr1[LANE:]], b_rel1 * 0.0, r1,
                       relu=True, out_fold=4)

    r2 = _fused_matmul([h4], [W_root2.T], b_rel2, None, relu=False,
                       lhs_fold=[4])
    b_chunks = _make_segsum(4)(h4, src_p, dst_p, zeros)
    Wr2 = W_rel2.T
    out = _fused_matmul(
        list(b_chunks),
        [Wr2[i * LANE : (i + 1) * LANE] for i in range(4)],
        b_rel2 * 0.0,
        r2,
        relu=False,
    )
    return out


# speedup vs baseline: 6.9786x; 1.0701x over previous
import functools

import jax
import jax.numpy as jnp
from jax import lax
from jax.experimental import pallas as pl
from jax.experimental.pallas import tpu as pltpu
from jax.experimental.pallas import tpu_sc as plsc

N = 10000
E = 160000
LANE = 128
E_PAD = 163840
IDX_ROWS = E_PAD // LANE
TILES = 16
ROWS_PER_TILE = IDX_ROWS // TILES
UNIT = 16
N_UNITS = ROWS_PER_TILE // UNIT
NBUF = 4
SLOT = 64
SLOTS = 2 * UNIT
STRIPE = 624
EXTRA = N - TILES * STRIPE
JUNK = 8
BM = 1000
VEC = 16


def _make_segsum(n_chunks):
    half_ch = n_chunks // 2
    mesh = plsc.VectorSubcoreMesh(core_axis_name="c", subcore_axis_name="s")

    @functools.partial(
        pl.kernel,
        mesh=mesh,
        out_type=[jax.ShapeDtypeStruct((N, LANE), jnp.float32)] * n_chunks,
        scratch_types=[
            pltpu.VMEM((UNIT, LANE), jnp.int32),
            pltpu.VMEM((SLOTS, SLOT), jnp.int32),
            pltpu.VMEM((SLOTS, SLOT), jnp.int32),
            pltpu.VMEM((NBUF, SLOT, LANE), jnp.float32),
            pltpu.VMEM_SHARED((N + JUNK, LANE), jnp.float32),
            pltpu.SemaphoreType.DMA((NBUF,)),
            pltpu.SemaphoreType.DMA((NBUF,)),
        ],
    )
    def segsum(xf, src_r, dst_r, zero_r, *rest):
        outs = rest[:n_chunks]
        sbuf, sidx, didx, rows, acc, gsem, ssem = rest[n_chunks:]

        c = lax.axis_index("c")
        s = lax.axis_index("s")

        n_grp = SLOTS // NBUF

        def process(ch, out_ref):
            pltpu.sync_copy(zero_r, acc.at[pl.ds(s * STRIPE, STRIPE)])

            @pl.when(s == TILES - 1)
            def _():
                pltpu.sync_copy(
                    zero_r.at[pl.ds(0, EXTRA)],
                    acc.at[pl.ds(TILES * STRIPE, EXTRA)],
                )

            plsc.subcore_barrier()

            def gather(j, b):
                pltpu.async_copy(
                    xf.at[sidx.at[j]], rows.at[b], gsem.at[b]
                )

            def gather_wait(j, b):
                pltpu.make_async_copy(
                    xf.at[sidx.at[j]], rows.at[b], gsem.at[b]
                ).wait()

            def scatter(j, b):
                pltpu.async_copy(
                    rows.at[b], acc.at[didx.at[j]], ssem.at[b], add=True
                )

            def scatter_wait(j, b):
                pltpu.make_async_copy(
                    rows.at[b], acc.at[didx.at[j]], ssem.at[b]
                ).wait()

            for u in range(N_UNITS):
                rb = s * ROWS_PER_TILE + u * UNIT
                pltpu.sync_copy(src_r.at[pl.ds(rb, UNIT)], sbuf)

                def idxt_s(r, carry):
                    for v in range(LANE // VEC):
                        dst_sl = pl.ds((v % 4) * VEC, VEC)
                        sidx[2 * r + v // 4, dst_sl] = (
                            sbuf[r, pl.ds(v * VEC, VEC)] * n_chunks + ch
                        )
                    return carry

                lax.fori_loop(0, UNIT, idxt_s, 0)
                pltpu.sync_copy(dst_r.at[pl.ds(rb, UNIT)], sbuf)

                def idxt_d(r, carry):
                    for v in range(LANE // VEC):
                        dst_sl = pl.ds((v % 4) * VEC, VEC)
                        didx[2 * r + v // 4, dst_sl] = sbuf[r, pl.ds(v * VEC, VEC)]
                    return carry

                lax.fori_loop(0, UNIT, idxt_d, 0)

                for b in range(NBUF - 1):
                    gather(b, b)

                def grp(g, carry):
                    j0 = NBUF * g
                    for k in range(NBUF):
                        j = j0 + k
                        gather_wait(j, k)
                        scatter(j, k)

                        if k == 0:
                            @pl.when(g > 0)
                            def _():
                                scatter_wait(j0 - 1, NBUF - 1)
                        else:
                            scatter_wait(j - 1, k - 1)

                        @pl.when(j < SLOTS - NBUF + 1)
                        def _(j=j, k=k):
                            gather(j + NBUF - 1, (k + NBUF - 1) % NBUF)

                    return carry

                lax.fori_loop(0, n_grp, grp, 0)
                scatter_wait(SLOTS - 1, NBUF - 1)
            plsc.subcore_barrier()
            pltpu.sync_copy(
                acc.at[pl.ds(s * STRIPE, STRIPE)],
                out_ref.at[pl.ds(s * STRIPE, STRIPE)],
            )

            @pl.when(s == TILES - 1)
            def _():
                pltpu.sync_copy(
                    acc.at[pl.ds(TILES * STRIPE, EXTRA)],
                    out_ref.at[pl.ds(TILES * STRIPE, EXTRA)],
                )

        for cid in range(2):
            for j in range(half_ch):
                ch = cid * half_ch + j

                @pl.when(c == cid)
                def _(ch=ch):
                    process(ch, outs[ch])

    return segsum


def _fused_matmul(lhs_list, rhs_list, bias, residual, relu,
                  lhs_fold=None, out_fold=1):
    n_in = len(lhs_list)
    lhs_fold = lhs_fold or [1] * n_in
    m = lhs_list[0].shape[0] // lhs_fold[0]
    n_out = rhs_list[0].shape[1]
    grid = (m // BM,)
    has_res = residual is not None

    def body(*refs):
        ls = refs[:n_in]
        rs = refs[n_in : 2 * n_in]
        b = refs[2 * n_in]
        res = refs[2 * n_in + 1] if has_res else None
        out = refs[-1]
        acc = b[...]
        if has_res:
            acc = acc + res[...]
        for i in range(n_in):
            lv = ls[i][...]
            if lhs_fold[i] > 1:
                lv = lv.reshape(BM, LANE * lhs_fold[i])
            acc = acc + jnp.dot(
                lv, rs[i][...], preferred_element_type=jnp.float32
            )
        if relu:
            acc = jnp.maximum(acc, 0.0)
        if out_fold > 1:
            out[...] = acc.reshape(out_fold * BM, LANE)
        else:
            out[...] = acc

    in_specs = (
        [
            pl.BlockSpec((f * BM, l.shape[1]), lambda mi: (mi, 0))
            for l, f in zip(lhs_list, lhs_fold)
        ]
        + [pl.BlockSpec(r.shape, lambda mi: (0, 0)) for r in rhs_list]
        + [pl.BlockSpec((1, n_out), lambda mi: (0, 0))]
    )
    args = list(lhs_list) + list(rhs_list) + [bias.reshape(1, n_out)]
    if has_res:
        in_specs.append(pl.BlockSpec((BM, n_out), lambda mi: (mi, 0)))
        args.append(residual)
    if out_fold > 1:
        out_specs = pl.BlockSpec((out_fold * BM, LANE), lambda mi: (mi, 0))
        out_shape = jax.ShapeDtypeStruct((out_fold * m, LANE), jnp.float32)
    else:
        out_specs = pl.BlockSpec((BM, n_out), lambda mi: (mi, 0))
        out_shape = jax.ShapeDtypeStruct((m, n_out), jnp.float32)
    return pl.pallas_call(
        body,
        grid=grid,
        in_specs=in_specs,
        out_specs=out_specs,
        out_shape=out_shape,
    )(*args)


def kernel(x, edge_attr, W_rel1, b_rel1, W_root1, W_rel2, b_rel2, W_root2,
           edge_index):
    del edge_attr
    src = edge_index[0].astype(jnp.int32)
    dst = edge_index[1].astype(jnp.int32)
    pad = E_PAD - E
    pad_i = jnp.arange(pad, dtype=jnp.int32)
    src_p = jnp.concatenate([src, pad_i % N]).reshape(IDX_ROWS, LANE)
    dst_p = jnp.concatenate([dst, N + (pad_i % JUNK)]).reshape(IDX_ROWS, LANE)
    zeros = jnp.zeros((STRIPE, LANE), jnp.float32)

    r1 = _fused_matmul([x], [W_root1.T], b_rel1, None, relu=False)
    a0, a1 = _make_segsum(2)(x.reshape(2 * N, LANE), src_p, dst_p, zeros)
    Wr1 = W_rel1.T
    h4 = _fused_matmul([a0, a1], [Wr1[:LANE], Wr1[LANE:]], b_rel1 * 0.0, r1,
                       relu=True, out_fold=4)

    r2 = _fused_matmul([h4], [W_root2.T], b_rel2, None, relu=False,
                       lhs_fold=[4])
    b_chunks = _make_segsum(4)(h4, src_p, dst_p, zeros)
    Wr2 = W_rel2.T
    out = _fused_matmul(
        list(b_chunks),
        [Wr2[i * LANE : (i + 1) * LANE] for i in range(4)],
        b_rel2 * 0.0,
        r2,
        relu=False,
    )
    return out


# speedup vs baseline: 7.0029x; 1.0035x over previous
import functools

import jax
import jax.numpy as jnp
import numpy as np
from jax import lax
from jax.experimental import pallas as pl
from jax.experimental.pallas import tpu as pltpu
from jax.experimental.pallas import tpu_sc as plsc

N = 10000
E = 160000
LANE = 128
E_PAD = 163840
IDX_ROWS = E_PAD // LANE
TILES = 16
ROWS_PER_TILE = IDX_ROWS // TILES
UNIT = 16
N_UNITS = ROWS_PER_TILE // UNIT
NBUF = 4
SLOT = 64
SLOTS = 2 * UNIT
STRIPE = 624
EXTRA = N - TILES * STRIPE
JUNK = 8
BM = 1000
VEC = 16

_PAD_SRC = np.arange(E_PAD - E, dtype=np.int32) % N
_PAD_DST = N + np.arange(E_PAD - E, dtype=np.int32) % JUNK


def _make_segsum(n_chunks):
    half_ch = n_chunks // 2
    mesh = plsc.VectorSubcoreMesh(core_axis_name="c", subcore_axis_name="s")

    @functools.partial(
        pl.kernel,
        mesh=mesh,
        out_type=[jax.ShapeDtypeStruct((N, LANE), jnp.float32)] * n_chunks,
        scratch_types=[
            pltpu.VMEM((UNIT, LANE), jnp.int32),
            pltpu.VMEM((SLOTS, SLOT), jnp.int32),
            pltpu.VMEM((SLOTS, SLOT), jnp.int32),
            pltpu.VMEM((NBUF, SLOT, LANE), jnp.float32),
            pltpu.VMEM_SHARED((N + JUNK, LANE), jnp.float32),
            pltpu.SemaphoreType.DMA((NBUF,)),
            pltpu.SemaphoreType.DMA((NBUF,)),
        ],
    )
    def segsum(xf, src_r, dst_r, zero_r, *rest):
        outs = rest[:n_chunks]
        sbuf, sidx, didx, rows, acc, gsem, ssem = rest[n_chunks:]

        c = lax.axis_index("c")
        s = lax.axis_index("s")

        n_grp = SLOTS // NBUF

        def process(ch, out_ref):
            pltpu.sync_copy(zero_r, acc.at[pl.ds(s * STRIPE, STRIPE)])

            @pl.when(s == TILES - 1)
            def _():
                pltpu.sync_copy(
                    zero_r.at[pl.ds(0, EXTRA)],
                    acc.at[pl.ds(TILES * STRIPE, EXTRA)],
                )

            plsc.subcore_barrier()

            def gather(j, b):
                pltpu.async_copy(
                    xf.at[sidx.at[j]], rows.at[b], gsem.at[b]
                )

            def gather_wait(j, b):
                pltpu.make_async_copy(
                    xf.at[sidx.at[j]], rows.at[b], gsem.at[b]
                ).wait()

            def scatter(j, b):
                pltpu.async_copy(
                    rows.at[b], acc.at[didx.at[j]], ssem.at[b], add=True
                )

            def scatter_wait(j, b):
                pltpu.make_async_copy(
                    rows.at[b], acc.at[didx.at[j]], ssem.at[b]
                ).wait()

            for u in range(N_UNITS):
                rb = s * ROWS_PER_TILE + u * UNIT
                pltpu.sync_copy(src_r.at[pl.ds(rb, UNIT)], sbuf)

                def idxt_s(r, carry):
                    for v in range(LANE // VEC):
                        dst_sl = pl.ds((v % 4) * VEC, VEC)
                        sidx[2 * r + v // 4, dst_sl] = (
                            sbuf[r, pl.ds(v * VEC, VEC)] * n_chunks + ch
                        )
                    return carry

                lax.fori_loop(0, UNIT, idxt_s, 0)
                pltpu.sync_copy(dst_r.at[pl.ds(rb, UNIT)], sbuf)

                def idxt_d(r, carry):
                    for v in range(LANE // VEC):
                        dst_sl = pl.ds((v % 4) * VEC, VEC)
                        didx[2 * r + v // 4, dst_sl] = sbuf[r, pl.ds(v * VEC, VEC)]
                    return carry

                lax.fori_loop(0, UNIT, idxt_d, 0)

                for b in range(NBUF - 1):
                    gather(b, b)

                def grp(g, carry):
                    j0 = NBUF * g
                    for k in range(NBUF):
                        j = j0 + k
                        gather_wait(j, k)
                        scatter(j, k)

                        if k == 0:
                            @pl.when(g > 0)
                            def _():
                                scatter_wait(j0 - 1, NBUF - 1)
                        else:
                            scatter_wait(j - 1, k - 1)

                        @pl.when(j < SLOTS - NBUF + 1)
                        def _(j=j, k=k):
                            gather(j + NBUF - 1, (k + NBUF - 1) % NBUF)

                    return carry

                lax.fori_loop(0, n_grp, grp, 0)
                scatter_wait(SLOTS - 1, NBUF - 1)
            plsc.subcore_barrier()
            pltpu.sync_copy(
                acc.at[pl.ds(s * STRIPE, STRIPE)],
                out_ref.at[pl.ds(s * STRIPE, STRIPE)],
            )

            @pl.when(s == TILES - 1)
            def _():
                pltpu.sync_copy(
                    acc.at[pl.ds(TILES * STRIPE, EXTRA)],
                    out_ref.at[pl.ds(TILES * STRIPE, EXTRA)],
                )

        for cid in range(2):
            for j in range(half_ch):
                ch = cid * half_ch + j

                @pl.when(c == cid)
                def _(ch=ch):
                    process(ch, outs[ch])

    return segsum


def _fused_matmul(lhs_list, rhs_list, bias, residual, relu,
                  lhs_fold=None, out_fold=1):
    n_in = len(lhs_list)
    lhs_fold = lhs_fold or [1] * n_in
    m = lhs_list[0].shape[0] // lhs_fold[0]
    n_out = rhs_list[0].shape[1]
    grid = (m // BM,)
    has_res = residual is not None

    def body(*refs):
        ls = refs[:n_in]
        rs = refs[n_in : 2 * n_in]
        b = refs[2 * n_in]
        res = refs[2 * n_in + 1] if has_res else None
        out = refs[-1]
        acc = b[...]
        if has_res:
            acc = acc + res[...]
        for i in range(n_in):
            lv = ls[i][...]
            if lhs_fold[i] > 1:
                lv = lv.reshape(BM, LANE * lhs_fold[i])
            acc = acc + jnp.dot(
                lv, rs[i][...], preferred_element_type=jnp.float32
            )
        if relu:
            acc = jnp.maximum(acc, 0.0)
        if out_fold > 1:
            out[...] = acc.reshape(out_fold * BM, LANE)
        else:
            out[...] = acc

    in_specs = (
        [
            pl.BlockSpec((f * BM, l.shape[1]), lambda mi: (mi, 0))
            for l, f in zip(lhs_list, lhs_fold)
        ]
        + [pl.BlockSpec(r.shape, lambda mi: (0, 0)) for r in rhs_list]
        + [pl.BlockSpec((1, n_out), lambda mi: (0, 0))]
    )
    args = list(lhs_list) + list(rhs_list) + [bias.reshape(1, n_out)]
    if has_res:
        in_specs.append(pl.BlockSpec((BM, n_out), lambda mi: (mi, 0)))
        args.append(residual)
    if out_fold > 1:
        out_specs = pl.BlockSpec((out_fold * BM, LANE), lambda mi: (mi, 0))
        out_shape = jax.ShapeDtypeStruct((out_fold * m, LANE), jnp.float32)
    else:
        out_specs = pl.BlockSpec((BM, n_out), lambda mi: (mi, 0))
        out_shape = jax.ShapeDtypeStruct((m, n_out), jnp.float32)
    return pl.pallas_call(
        body,
        grid=grid,
        in_specs=in_specs,
        out_specs=out_specs,
        out_shape=out_shape,
    )(*args)


def kernel(x, edge_attr, W_rel1, b_rel1, W_root1, W_rel2, b_rel2, W_root2,
           edge_index):
    del edge_attr
    src = edge_index[0].astype(jnp.int32)
    dst = edge_index[1].astype(jnp.int32)
    src_p = jnp.concatenate([src, _PAD_SRC]).reshape(IDX_ROWS, LANE)
    dst_p = jnp.concatenate([dst, _PAD_DST]).reshape(IDX_ROWS, LANE)
    zeros = jnp.zeros((STRIPE, LANE), jnp.float32)

    r1 = _fused_matmul([x], [W_root1.T], b_rel1, None, relu=False)
    a0, a1 = _make_segsum(2)(x.reshape(2 * N, LANE), src_p, dst_p, zeros)
    Wr1 = W_rel1.T
    h4 = _fused_matmul([a0, a1], [Wr1[:LANE], Wr1[LANE:]], b_rel1 * 0.0, r1,
                       relu=True, out_fold=4)

    r2 = _fused_matmul([h4], [W_root2.T], b_rel2, None, relu=False,
                       lhs_fold=[4])
    b_chunks = _make_segsum(4)(h4, src_p, dst_p, zeros)
    Wr2 = W_rel2.T
    out = _fused_matmul(
        list(b_chunks),
        [Wr2[i * LANE : (i + 1) * LANE] for i in range(4)],
        b_rel2 * 0.0,
        r2,
        relu=False,
    )
    return out


# speedup vs baseline: 7.1306x; 1.0182x over previous
import functools

import jax
import jax.numpy as jnp
import numpy as np
from jax import lax
from jax.experimental import pallas as pl
from jax.experimental.pallas import tpu as pltpu
from jax.experimental.pallas import tpu_sc as plsc

N = 10000
E = 160000
LANE = 128
E_PAD = 163840
IDX_ROWS = E_PAD // LANE
TILES = 16
ROWS_PER_TILE = IDX_ROWS // TILES
UNIT = 8
N_UNITS = ROWS_PER_TILE // UNIT
NBUF = 4
SLOT = 64
USLOTS = 2 * UNIT
SLOTS = N_UNITS * USLOTS
STRIPE = 624
EXTRA = N - TILES * STRIPE
JUNK = 8
BM = 1000
VEC = 16

_PAD_SRC = np.arange(E_PAD - E, dtype=np.int32) % N
_PAD_DST = N + np.arange(E_PAD - E, dtype=np.int32) % JUNK


def _make_segsum(n_chunks):
    half_ch = n_chunks // 2
    mesh = plsc.VectorSubcoreMesh(core_axis_name="c", subcore_axis_name="s")

    @functools.partial(
        pl.kernel,
        mesh=mesh,
        out_type=[jax.ShapeDtypeStruct((N, LANE), jnp.float32)] * n_chunks,
        scratch_types=[
            pltpu.VMEM((UNIT, LANE), jnp.int32),
            pltpu.VMEM((2, USLOTS, SLOT), jnp.int32),
            pltpu.VMEM((2, USLOTS, SLOT), jnp.int32),
            pltpu.VMEM((NBUF, SLOT, LANE), jnp.float32),
            pltpu.VMEM_SHARED((N + JUNK, LANE), jnp.float32),
            pltpu.SemaphoreType.DMA((NBUF,)),
            pltpu.SemaphoreType.DMA((NBUF,)),
        ],
    )
    def segsum(xf, src_r, dst_r, zero_r, *rest):
        outs = rest[:n_chunks]
        sbuf, sidx, didx, rows, acc, gsem, ssem = rest[n_chunks:]

        c = lax.axis_index("c")
        s = lax.axis_index("s")

        n_grp = SLOTS // NBUF

        def process(ch, out_ref):
            pltpu.sync_copy(zero_r, acc.at[pl.ds(s * STRIPE, STRIPE)])

            @pl.when(s == TILES - 1)
            def _():
                pltpu.sync_copy(
                    zero_r.at[pl.ds(0, EXTRA)],
                    acc.at[pl.ds(TILES * STRIPE, EXTRA)],
                )

            plsc.subcore_barrier()

            def gather(j, b):
                pltpu.async_copy(
                    xf.at[sidx.at[(j // USLOTS) % 2, j % USLOTS]],
                    rows.at[b], gsem.at[b],
                )

            def gather_wait(j, b):
                pltpu.make_async_copy(
                    xf.at[sidx.at[(j // USLOTS) % 2, j % USLOTS]],
                    rows.at[b], gsem.at[b],
                ).wait()

            def scatter(j, b):
                pltpu.async_copy(
                    rows.at[b],
                    acc.at[didx.at[(j // USLOTS) % 2, j % USLOTS]],
                    ssem.at[b], add=True,
                )

            def scatter_wait(j, b):
                pltpu.make_async_copy(
                    rows.at[b],
                    acc.at[didx.at[(j // USLOTS) % 2, j % USLOTS]],
                    ssem.at[b],
                ).wait()

            def stage(u):
                p = u % 2
                rb = s * ROWS_PER_TILE + u * UNIT
                pltpu.sync_copy(src_r.at[pl.ds(rb, UNIT)], sbuf)

                def idxt_s(r, carry):
                    for v in range(LANE // VEC):
                        dst_sl = pl.ds((v % 4) * VEC, VEC)
                        sidx[p, 2 * r + v // 4, dst_sl] = (
                            sbuf[r, pl.ds(v * VEC, VEC)] * n_chunks + ch
                        )
                    return carry

                lax.fori_loop(0, UNIT, idxt_s, 0)
                pltpu.sync_copy(dst_r.at[pl.ds(rb, UNIT)], sbuf)

                def idxt_d(r, carry):
                    for v in range(LANE // VEC):
                        dst_sl = pl.ds((v % 4) * VEC, VEC)
                        didx[p, 2 * r + v // 4, dst_sl] = sbuf[
                            r, pl.ds(v * VEC, VEC)
                        ]
                    return carry

                lax.fori_loop(0, UNIT, idxt_d, 0)

            stage(0)
            for b in range(NBUF - 1):
                gather(b, b)

            def grp(g, carry):
                j0 = NBUF * g
                for k in range(NBUF):
                    j = j0 + k
                    gather_wait(j, k)
                    scatter(j, k)

                    if k == 0:
                        @pl.when(g > 0)
                        def _():
                            scatter_wait(j0 - 1, NBUF - 1)

                        @pl.when(lax.rem(g, NBUF) == 0)
                        def _():
                            @pl.when(g < n_grp - NBUF)
                            def _():
                                stage(g // NBUF + 1)
                    else:
                        scatter_wait(j - 1, k - 1)

                    @pl.when(j < SLOTS - NBUF + 1)
                    def _(j=j, k=k):
                        gather(j + NBUF - 1, (k + NBUF - 1) % NBUF)

                return carry

            lax.fori_loop(0, n_grp, grp, 0)
            scatter_wait(SLOTS - 1, NBUF - 1)
            plsc.subcore_barrier()
            pltpu.sync_copy(
                acc.at[pl.ds(s * STRIPE, STRIPE)],
                out_ref.at[pl.ds(s * STRIPE, STRIPE)],
            )

            @pl.when(s == TILES - 1)
            def _():
                pltpu.sync_copy(
                    acc.at[pl.ds(TILES * STRIPE, EXTRA)],
                    out_ref.at[pl.ds(TILES * STRIPE, EXTRA)],
                )

        for cid in range(2):
            for j in range(half_ch):
                ch = cid * half_ch + j

                @pl.when(c == cid)
                def _(ch=ch):
                    process(ch, outs[ch])

    return segsum


def _fused_matmul(lhs_list, rhs_list, bias, residual, relu,
                  lhs_fold=None, out_fold=1):
    n_in = len(lhs_list)
    lhs_fold = lhs_fold or [1] * n_in
    m = lhs_list[0].shape[0] // lhs_fold[0]
    n_out = rhs_list[0].shape[1]
    grid = (m // BM,)
    has_res = residual is not None

    def body(*refs):
        ls = refs[:n_in]
        rs = refs[n_in : 2 * n_in]
        b = refs[2 * n_in]
        res = refs[2 * n_in + 1] if has_res else None
        out = refs[-1]
        acc = b[...]
        if has_res:
            acc = acc + res[...]
        for i in range(n_in):
            lv = ls[i][...]
            if lhs_fold[i] > 1:
                lv = lv.reshape(BM, LANE * lhs_fold[i])
            acc = acc + jnp.dot(
                lv, rs[i][...], preferred_element_type=jnp.float32
            )
        if relu:
            acc = jnp.maximum(acc, 0.0)
        if out_fold > 1:
            out[...] = acc.reshape(out_fold * BM, LANE)
        else:
            out[...] = acc

    in_specs = (
        [
            pl.BlockSpec((f * BM, l.shape[1]), lambda mi: (mi, 0))
            for l, f in zip(lhs_list, lhs_fold)
        ]
        + [pl.BlockSpec(r.shape, lambda mi: (0, 0)) for r in rhs_list]
        + [pl.BlockSpec((1, n_out), lambda mi: (0, 0))]
    )
    args = list(lhs_list) + list(rhs_list) + [bias.reshape(1, n_out)]
    if has_res:
        in_specs.append(pl.BlockSpec((BM, n_out), lambda mi: (mi, 0)))
        args.append(residual)
    if out_fold > 1:
        out_specs = pl.BlockSpec((out_fold * BM, LANE), lambda mi: (mi, 0))
        out_shape = jax.ShapeDtypeStruct((out_fold * m, LANE), jnp.float32)
    else:
        out_specs = pl.BlockSpec((BM, n_out), lambda mi: (mi, 0))
        out_shape = jax.ShapeDtypeStruct((m, n_out), jnp.float32)
    return pl.pallas_call(
        body,
        grid=grid,
        in_specs=in_specs,
        out_specs=out_specs,
        out_shape=out_shape,
    )(*args)


def kernel(x, edge_attr, W_rel1, b_rel1, W_root1, W_rel2, b_rel2, W_root2,
           edge_index):
    del edge_attr
    src = edge_index[0].astype(jnp.int32)
    dst = edge_index[1].astype(jnp.int32)
    src_p = jnp.concatenate([src, _PAD_SRC]).reshape(IDX_ROWS, LANE)
    dst_p = jnp.concatenate([dst, _PAD_DST]).reshape(IDX_ROWS, LANE)
    zeros = jnp.zeros((STRIPE, LANE), jnp.float32)

    r1 = _fused_matmul([x], [W_root1.T], b_rel1, None, relu=False)
    a0, a1 = _make_segsum(2)(x.reshape(2 * N, LANE), src_p, dst_p, zeros)
    Wr1 = W_rel1.T
    h4 = _fused_matmul([a0, a1], [Wr1[:LANE], Wr1[LANE:]], b_rel1 * 0.0, r1,
                       relu=True, out_fold=4)

    r2 = _fused_matmul([h4], [W_root2.T], b_rel2, None, relu=False,
                       lhs_fold=[4])
    b_chunks = _make_segsum(4)(h4, src_p, dst_p, zeros)
    Wr2 = W_rel2.T
    out = _fused_matmul(
        list(b_chunks),
        [Wr2[i * LANE : (i + 1) * LANE] for i in range(4)],
        b_rel2 * 0.0,
        r2,
        relu=False,
    )
    return out
